# baseline (speedup 1.0000x reference)
import jax
import jax.numpy as jnp
from jax import lax
from jax.experimental import pallas as pl
from jax.experimental.pallas import tpu as pltpu
from jax.experimental.pallas import tpu_sc as plsc

BATCH = 4096
HIST = 200
EMB = 128
TOTAL = BATCH * HIST

_info = plsc.get_sparse_core_info()
NC = _info.num_cores
NS = _info.num_subcores
NW = NC * NS

CHUNK = 80
PER_W = TOTAL // NW
N_CHUNK = PER_W // CHUNK
NBUF = 4
LOOK = 2


def _gather_body(codes_hbm, table_hbm, out_hbm, idx_v, rows_v, sp, *sems):
    gsem = sems[:NBUF]
    csem = sems[NBUF:2 * NBUF]
    wsem = sems[2 * NBUF:]
    cid = lax.axis_index("c")
    sid = lax.axis_index("s")
    wid = sid * NC + cid
    base = wid * PER_W

    def fire_gather(b, g):
        off = base + g * CHUNK
        pltpu.sync_copy(codes_hbm.at[pl.ds(off, CHUNK)], idx_v.at[b])
        pltpu.async_copy(table_hbm.at[idx_v.at[b]], rows_v.at[b], gsem[b])

    def wait_gather(b, g):
        pltpu.make_async_copy(
            table_hbm.at[idx_v.at[b]], rows_v.at[b], gsem[b]
        ).wait()

    def fire_xfer(b, g):
        pltpu.async_copy(rows_v.at[b], sp.at[sid, b], csem[b])

    def wait_xfer(b, g):
        pltpu.make_async_copy(rows_v.at[b], sp.at[sid, b], csem[b]).wait()

    def fire_write(b, g):
        off = base + g * CHUNK
        pltpu.async_copy(sp.at[sid, b], out_hbm.at[pl.ds(off, CHUNK)], wsem[b])

    def wait_write(b, g):
        off = base + g * CHUNK
        pltpu.make_async_copy(
            sp.at[sid, b], out_hbm.at[pl.ds(off, CHUNK)], wsem[b]
        ).wait()

    def slot(g, b, have_write_drain, have_prev_xfer, do_fire):
        wait_gather(b, g)
        if have_write_drain:
            wait_write(b, g - NBUF)
        fire_xfer(b, g)
        if have_prev_xfer:
            b1 = (b - 1) % NBUF
            wait_xfer(b1, g - 1)
            fire_write(b1, g - 1)
        if do_fire:
            fire_gather((b + LOOK) % NBUF, g + LOOK)

    for b in range(LOOK):
        fire_gather(b, b)

    for b in range(NBUF):
        slot(b, b, have_write_drain=False, have_prev_xfer=(b >= 1),
             do_fire=True)

    def outer(o, carry):
        for b in range(NBUF):
            slot(o * NBUF + b, b, have_write_drain=True, have_prev_xfer=True,
                 do_fire=True)
        return carry

    lax.fori_loop(1, N_CHUNK // NBUF - 1, outer, 0)

    for b in range(NBUF):
        g = N_CHUNK - NBUF + b
        slot(g, b, have_write_drain=True, have_prev_xfer=True,
             do_fire=(g + LOOK < N_CHUNK))
    bl = (N_CHUNK - 1) % NBUF
    wait_xfer(bl, N_CHUNK - 1)
    fire_write(bl, N_CHUNK - 1)
    for b in range(NBUF):
        wait_write(b, N_CHUNK - NBUF + b)


@jax.jit
def kernel(codes, table):
    codes_flat = codes.reshape(TOTAL).astype(jnp.int32)
    mesh = plsc.VectorSubcoreMesh(core_axis_name="c", subcore_axis_name="s")
    k = pl.kernel(
        _gather_body,
        mesh=mesh,
        out_type=jax.ShapeDtypeStruct((TOTAL, EMB), jnp.float32),
        scratch_types=(
            [
                pltpu.VMEM((NBUF, CHUNK), jnp.int32),
                pltpu.VMEM((NBUF, CHUNK, EMB), jnp.float32),
                pltpu.MemorySpace.VMEM_SHARED((NS, NBUF, CHUNK, EMB),
                                              jnp.float32),
            ]
            + [pltpu.SemaphoreType.DMA] * (3 * NBUF)
        ),
    )
    out = k(codes_flat, table)
    return out.reshape(BATCH, HIST, EMB)


# speedup vs baseline: 1.0909x; 1.0909x over previous
import jax
import jax.numpy as jnp
from jax import lax
from jax.experimental import pallas as pl
from jax.experimental.pallas import tpu as pltpu
from jax.experimental.pallas import tpu_sc as plsc

BATCH = 4096
HIST = 200
EMB = 128
TOTAL = BATCH * HIST

_info = plsc.get_sparse_core_info()
NC = _info.num_cores
NS = _info.num_subcores
NW = NC * NS

CHUNK = 128
PER_W = TOTAL // NW
N_CHUNK = PER_W // CHUNK
NBUF = 5
LOOK = 2


def _gather_body(codes_hbm, table_hbm, out_hbm, idx_all, rows_v, *sems):
    gsem = sems[:NBUF]
    wsem = sems[NBUF:]
    wid = lax.axis_index("s") * NC + lax.axis_index("c")
    base = wid * PER_W

    pltpu.sync_copy(codes_hbm.at[pl.ds(base, PER_W)], idx_all)

    def idx_slice(g):
        return idx_all.at[pl.ds(g * CHUNK, CHUNK)]

    def fire_gather(b, g):
        pltpu.async_copy(table_hbm.at[idx_slice(g)], rows_v.at[b], gsem[b])

    def wait_gather(b, g):
        pltpu.make_async_copy(
            table_hbm.at[idx_slice(g)], rows_v.at[b], gsem[b]
        ).wait()

    def fire_write(b, g):
        off = base + g * CHUNK
        pltpu.async_copy(rows_v.at[b], out_hbm.at[pl.ds(off, CHUNK)], wsem[b])

    def wait_write(b, g):
        off = base + g * CHUNK
        pltpu.make_async_copy(
            rows_v.at[b], out_hbm.at[pl.ds(off, CHUNK)], wsem[b]
        ).wait()

    def slot(g, b, do_drain, do_fire):
        wait_gather(b, g)
        fire_write(b, g)
        if do_drain:
            wait_write((b + LOOK) % NBUF, g + LOOK - NBUF)
        if do_fire:
            fire_gather((b + LOOK) % NBUF, g + LOOK)

    for b in range(LOOK):
        fire_gather(b, b)

    for b in range(NBUF):
        slot(b, b, do_drain=(b >= NBUF - LOOK), do_fire=True)

    def outer(o, carry):
        for b in range(NBUF):
            slot(o * NBUF + b, b, do_drain=True, do_fire=True)
        return carry

    lax.fori_loop(1, N_CHUNK // NBUF - 1, outer, 0)

    for b in range(NBUF):
        g = N_CHUNK - NBUF + b
        slot(g, b, do_drain=(g + LOOK < N_CHUNK), do_fire=(g + LOOK < N_CHUNK))
    for b in range(NBUF):
        wait_write(b, N_CHUNK - NBUF + b)


@jax.jit
def kernel(codes, table):
    codes_flat = codes.reshape(TOTAL).astype(jnp.int32)
    mesh = plsc.VectorSubcoreMesh(core_axis_name="c", subcore_axis_name="s")
    k = pl.kernel(
        _gather_body,
        mesh=mesh,
        out_type=jax.ShapeDtypeStruct((TOTAL, EMB), jnp.float32),
        scratch_types=(
            [
                pltpu.VMEM((PER_W,), jnp.int32),
                pltpu.VMEM((NBUF, CHUNK, EMB), jnp.float32),
            ]
            + [pltpu.SemaphoreType.DMA] * (2 * NBUF)
        ),
    )
    out = k(codes_flat, table)
    return out.reshape(BATCH, HIST, EMB)


# speedup vs baseline: 1.0958x; 1.0044x over previous
import jax
import jax.numpy as jnp
from jax import lax
from jax.experimental import pallas as pl
from jax.experimental.pallas import tpu as pltpu
from jax.experimental.pallas import tpu_sc as plsc

BATCH = 4096
HIST = 200
EMB = 128
TOTAL = BATCH * HIST

_info = plsc.get_sparse_core_info()
NC = _info.num_cores
NS = _info.num_subcores
NW = NC * NS

CHUNK = 128
PER_W = TOTAL // NW
N_CHUNK = PER_W // CHUNK
NBUF = 5
LOOK = 3


def _gather_body(codes_hbm, table_hbm, out_hbm, idx_all, rows_v, *sems):
    gsem = sems[:NBUF]
    wsem = sems[NBUF:]
    wid = lax.axis_index("s") * NC + lax.axis_index("c")
    base = wid * PER_W

    pltpu.sync_copy(codes_hbm.at[pl.ds(base, PER_W)], idx_all)

    def idx_slice(g):
        return idx_all.at[pl.ds(g * CHUNK, CHUNK)]

    def fire_gather(b, g):
        pltpu.async_copy(table_hbm.at[idx_slice(g)], rows_v.at[b], gsem[b])

    def wait_gather(b, g):
        pltpu.make_async_copy(
            table_hbm.at[idx_slice(g)], rows_v.at[b], gsem[b]
        ).wait()

    def fire_write(b, g):
        off = base + g * CHUNK
        pltpu.async_copy(rows_v.at[b], out_hbm.at[pl.ds(off, CHUNK)], wsem[b])

    def wait_write(b, g):
        off = base + g * CHUNK
        pltpu.make_async_copy(
            rows_v.at[b], out_hbm.at[pl.ds(off, CHUNK)], wsem[b]
        ).wait()

    def slot(g, b, do_drain, do_fire):
        wait_gather(b, g)
        fire_write(b, g)
        if do_drain:
            wait_write((b + LOOK) % NBUF, g + LOOK - NBUF)
        if do_fire:
            fire_gather((b + LOOK) % NBUF, g + LOOK)

    for b in range(LOOK):
        fire_gather(b, b)

    for b in range(NBUF):
        slot(b, b, do_drain=(b >= NBUF - LOOK), do_fire=True)

    def outer(o, carry):
        for b in range(NBUF):
            slot(o * NBUF + b, b, do_drain=True, do_fire=True)
        return carry

    lax.fori_loop(1, N_CHUNK // NBUF - 1, outer, 0)

    for b in range(NBUF):
        g = N_CHUNK - NBUF + b
        slot(g, b, do_drain=(g + LOOK < N_CHUNK), do_fire=(g + LOOK < N_CHUNK))
    for b in range(NBUF):
        wait_write(b, N_CHUNK - NBUF + b)


@jax.jit
def kernel(codes, table):
    codes_flat = codes.reshape(TOTAL).astype(jnp.int32)
    mesh = plsc.VectorSubcoreMesh(core_axis_name="c", subcore_axis_name="s")
    k = pl.kernel(
        _gather_body,
        mesh=mesh,
        out_type=jax.ShapeDtypeStruct((TOTAL, EMB), jnp.float32),
        scratch_types=(
            [
                pltpu.VMEM((PER_W,), jnp.int32),
                pltpu.VMEM((NBUF, CHUNK, EMB), jnp.float32),
            ]
            + [pltpu.SemaphoreType.DMA] * (2 * NBUF)
        ),
    )
    out = k(codes_flat, table)
    return out.reshape(BATCH, HIST, EMB)
